# Initial kernel scaffold; baseline (speedup 1.0000x reference)
import jax
import jax.numpy as jnp
from jax.experimental import pallas as pl


def kernel(in_feat, edge_index, W1, b1, W2, b2):
    raise NotImplementedError("write your pallas kernel here")


# speedup vs baseline: 16.2148x; 16.2148x over previous
import functools

import jax
import jax.numpy as jnp
from jax import lax
from jax.experimental import pallas as pl
from jax.experimental.pallas import tpu as pltpu
from jax.experimental.pallas import tpu_sc as plsc

N = 10000
E = 320000
IN_FEATS = 128
H_FEATS = 64
NUM_CLASSES = 16
K = 10
ALPHA = 0.1

NC = 2
NS = 16
NW = NC * NS
NPAD = 10240
RT = NPAD // NS
NB = 128
NNB = RT // NB
CH = 125
EW = E // NW
NCH = EW // CH
GC = 8
NG = NCH // GC
F32 = jnp.float32

_mesh = plsc.VectorSubcoreMesh(
    core_axis_name="c", subcore_axis_name="s", num_cores=NC, num_subcores=NS
)
_sc_params = pltpu.CompilerParams(use_tc_tiling_on_sc=False)


def _mlp_body(x_ref, w1_ref, b1_ref, w2_ref, b2_ref, o_ref):
    x = x_ref[...]
    h = lax.dot_general(x, w1_ref[...], (((1,), (1,)), ((), ())),
                        preferred_element_type=F32)
    h = jnp.maximum(h + b1_ref[...], 0.0)
    o = lax.dot_general(h, w2_ref[...], (((1,), (1,)), ((), ())),
                        preferred_element_type=F32)
    o_ref[...] = o + b2_ref[...]


def _mlp(in_feat, W1, b1, W2, b2):
    blk = 2000
    grid = (N // blk,)
    return pl.pallas_call(
        _mlp_body,
        grid=grid,
        in_specs=[
            pl.BlockSpec((blk, IN_FEATS), lambda i: (i, 0)),
            pl.BlockSpec((H_FEATS, IN_FEATS), lambda i: (0, 0)),
            pl.BlockSpec((1, H_FEATS), lambda i: (0, 0)),
            pl.BlockSpec((NUM_CLASSES, H_FEATS), lambda i: (0, 0)),
            pl.BlockSpec((1, NUM_CLASSES), lambda i: (0, 0)),
        ],
        out_specs=pl.BlockSpec((blk, NUM_CLASSES), lambda i: (i, 0)),
        out_shape=jax.ShapeDtypeStruct((N, NUM_CLASSES), F32),
    )(in_feat, W1, b1.reshape(1, H_FEATS), W2, b2.reshape(1, NUM_CLASSES))


def _deg_body(srcs, dsts, out, sidx, didx, ones, zb, degs_sp, degd_sp):
    c = lax.axis_index("c")
    s = lax.axis_index("s")
    wid = c * NS + s
    row0 = s * RT

    def fill(k, _):
        ones[k] = jnp.ones((16,), F32)
        zb[k] = jnp.zeros((16,), F32)
        return 0

    lax.fori_loop(0, NB, fill, 0)
    for kk in range(NNB):
        pltpu.sync_copy(zb, degs_sp.at[pl.ds(row0 + kk * NB, NB)])
        pltpu.sync_copy(zb, degd_sp.at[pl.ds(row0 + kk * NB, NB)])
    plsc.subcore_barrier()

    def gbody(g, _):
        pltpu.sync_copy(srcs.at[wid, pl.ds(g * GC, GC)], sidx)
        pltpu.sync_copy(dsts.at[wid, pl.ds(g * GC, GC)], didx)

        def ebody(j, _):
            pltpu.sync_copy(ones, degs_sp.at[sidx.at[j]], add=True)
            pltpu.sync_copy(ones, degd_sp.at[didx.at[j]], add=True)
            return 0

        lax.fori_loop(0, GC, ebody, 0)
        return 0

    lax.fori_loop(0, NG, gbody, 0)
    plsc.subcore_barrier()
    pltpu.sync_copy(degs_sp.at[pl.ds(row0, RT)], out.at[c, 0, pl.ds(row0, RT)])
    pltpu.sync_copy(degd_sp.at[pl.ds(row0, RT)], out.at[c, 1, pl.ds(row0, RT)])


_deg_kernel = functools.partial(
    pl.kernel,
    out_type=jax.ShapeDtypeStruct((NC, 2, NPAD, 16), F32),
    mesh=_mesh,
    compiler_params=_sc_params,
    scratch_types=[
        pltpu.VMEM((GC, CH), jnp.int32),
        pltpu.VMEM((GC, CH), jnp.int32),
        pltpu.VMEM((CH, 16), F32),
        pltpu.VMEM((NB, 16), F32),
        pltpu.VMEM_SHARED((NPAD, 16), F32),
        pltpu.VMEM_SHARED((NPAD, 16), F32),
    ],
)(_deg_body)


def _comb_body(d_ref, z_ref, nd_ref, ns_ref, g0_ref, a0_ref):
    ds = d_ref[0, 0] + d_ref[1, 0]
    dd = d_ref[0, 1] + d_ref[1, 1]
    z = z_ref[...]
    rows = lax.broadcasted_iota(jnp.int32, (NPAD, 16), 0)
    valid = rows < N
    ns_ref[...] = jnp.where(valid, lax.rsqrt(jnp.maximum(ds, 1.0)), 0.0)
    nd_ref[...] = jnp.where(
        valid, (1.0 - ALPHA) * lax.rsqrt(jnp.maximum(dd, 1.0)), 0.0)
    tail = jnp.zeros((NPAD - N, 16), F32)
    g0_ref[:N] = ALPHA * z
    g0_ref[N:] = tail
    a0_ref[:N] = z * jnp.sqrt(jnp.maximum(dd[:N], 1.0))
    a0_ref[N:] = tail


def _combine(degs, z0):
    shp = jax.ShapeDtypeStruct((NPAD, 16), F32)
    return pl.pallas_call(
        _comb_body,
        out_shape=(shp, shp, shp, shp),
    )(degs, z0)


def _prop_body(a0, a1, nd, ns, g0, srcs, dsts, out,
               sidx, didx, ba0, ba1, bnd, bns, bg0, bu, bz, gbuf, u_sp,
               acc_sp, sem):
    c = lax.axis_index("c")
    s = lax.axis_index("s")
    wid = c * NS + s
    row0 = s * RT

    def zfill(k, _):
        bz[k] = jnp.zeros((16,), F32)
        return 0

    lax.fori_loop(0, NB, zfill, 0)

    for kk in range(NNB):
        r0 = row0 + kk * NB
        pltpu.sync_copy(a0.at[pl.ds(r0, NB)], ba0)
        pltpu.sync_copy(a1.at[pl.ds(r0, NB)], ba1)
        pltpu.sync_copy(nd.at[pl.ds(r0, NB)], bnd)
        pltpu.sync_copy(ns.at[pl.ds(r0, NB)], bns)
        pltpu.sync_copy(g0.at[pl.ds(r0, NB)], bg0)

        def nbody(k, _):
            feat = bnd[k] * (ba0[k] + ba1[k]) + bg0[k]
            bu[k] = bns[k] * feat
            return 0

        lax.fori_loop(0, NB, nbody, 0)
        pltpu.sync_copy(bu, u_sp.at[pl.ds(r0, NB)])
        pltpu.sync_copy(bz, acc_sp.at[pl.ds(r0, NB)])

    plsc.subcore_barrier()

    def gbody(g, _):
        pltpu.sync_copy(srcs.at[wid, pl.ds(g * GC, GC)], sidx)
        pltpu.sync_copy(dsts.at[wid, pl.ds(g * GC, GC)], didx)

        def ebody(j, _):
            pltpu.async_copy(u_sp.at[sidx.at[j]], gbuf, sem).wait()
            pltpu.sync_copy(gbuf, acc_sp.at[didx.at[j]], add=True)
            return 0

        lax.fori_loop(0, GC, ebody, 0)
        return 0

    lax.fori_loop(0, NG, gbody, 0)
    plsc.subcore_barrier()
    pltpu.sync_copy(acc_sp.at[pl.ds(row0, RT)], out.at[c, pl.ds(row0, RT)])


_prop_kernel = functools.partial(
    pl.kernel,
    out_type=jax.ShapeDtypeStruct((NC, NPAD, 16), F32),
    mesh=_mesh,
    compiler_params=_sc_params,
    scratch_types=[
        pltpu.VMEM((GC, CH), jnp.int32),
        pltpu.VMEM((GC, CH), jnp.int32),
        pltpu.VMEM((NB, 16), F32),
        pltpu.VMEM((NB, 16), F32),
        pltpu.VMEM((NB, 16), F32),
        pltpu.VMEM((NB, 16), F32),
        pltpu.VMEM((NB, 16), F32),
        pltpu.VMEM((NB, 16), F32),
        pltpu.VMEM((NB, 16), F32),
        pltpu.VMEM((CH, 16), F32),
        pltpu.VMEM_SHARED((NPAD, 16), F32),
        pltpu.VMEM_SHARED((NPAD, 16), F32),
        pltpu.SemaphoreType.DMA,
    ],
)(_prop_body)


def _final_body(a0_ref, a1_ref, nd_ref, g0_ref, o_ref):
    o_ref[...] = (nd_ref[:N] * (a0_ref[:N] + a1_ref[:N]) + g0_ref[:N])


def _final(a0, a1, nd, g0):
    return pl.pallas_call(
        _final_body,
        out_shape=jax.ShapeDtypeStruct((N, 16), F32),
    )(a0, a1, nd, g0)


def kernel(in_feat, edge_index, W1, b1, W2, b2):
    edge_index = edge_index.astype(jnp.int32)
    srcs = edge_index[0].reshape(NW, NCH, CH)
    dsts = edge_index[1].reshape(NW, NCH, CH)

    z0 = _mlp(in_feat, W1, b1, W2, b2)
    degs = _deg_kernel(srcs, dsts)
    nd, ns, g0, a0 = _combine(degs, z0)
    a1 = jnp.zeros((NPAD, 16), F32)
    for _ in range(K):
        acc = _prop_kernel(a0, a1, nd, ns, g0, srcs, dsts)
        a0, a1 = acc[0], acc[1]
    return _final(a0, a1, nd, g0)[:N]


# speedup vs baseline: 16.2195x; 1.0003x over previous
import functools

import jax
import jax.numpy as jnp
from jax import lax
from jax.experimental import pallas as pl
from jax.experimental.pallas import tpu as pltpu
from jax.experimental.pallas import tpu_sc as plsc

N = 10000
E = 320000
IN_FEATS = 128
H_FEATS = 64
NUM_CLASSES = 16
K = 10
ALPHA = 0.1

NC = 2
NS = 16
NW = NC * NS
NPAD = 10240
RT = NPAD // NS
NB = 128
NNB = RT // NB
CH = 125
EW = E // NW
NCH = EW // CH
GC = 8
NG = NCH // GC
F32 = jnp.float32

_mesh = plsc.VectorSubcoreMesh(
    core_axis_name="c", subcore_axis_name="s", num_cores=NC, num_subcores=NS
)
_sc_params = pltpu.CompilerParams(use_tc_tiling_on_sc=False)


def _mlp_body(x_ref, w1_ref, b1_ref, w2_ref, b2_ref, o_ref):
    x = x_ref[...]
    h = lax.dot_general(x, w1_ref[...], (((1,), (1,)), ((), ())),
                        preferred_element_type=F32)
    h = jnp.maximum(h + b1_ref[...], 0.0)
    o = lax.dot_general(h, w2_ref[...], (((1,), (1,)), ((), ())),
                        preferred_element_type=F32)
    o_ref[...] = o + b2_ref[...]


def _mlp(in_feat, W1, b1, W2, b2):
    blk = 2000
    grid = (N // blk,)
    return pl.pallas_call(
        _mlp_body,
        grid=grid,
        in_specs=[
            pl.BlockSpec((blk, IN_FEATS), lambda i: (i, 0)),
            pl.BlockSpec((H_FEATS, IN_FEATS), lambda i: (0, 0)),
            pl.BlockSpec((1, H_FEATS), lambda i: (0, 0)),
            pl.BlockSpec((NUM_CLASSES, H_FEATS), lambda i: (0, 0)),
            pl.BlockSpec((1, NUM_CLASSES), lambda i: (0, 0)),
        ],
        out_specs=pl.BlockSpec((blk, NUM_CLASSES), lambda i: (i, 0)),
        out_shape=jax.ShapeDtypeStruct((N, NUM_CLASSES), F32),
    )(in_feat, W1, b1.reshape(1, H_FEATS), W2, b2.reshape(1, NUM_CLASSES))


def _deg_body(srcs, dsts, out, sidx, didx, ones, zb, degs_sp, degd_sp):
    c = lax.axis_index("c")
    s = lax.axis_index("s")
    wid = c * NS + s
    row0 = s * RT

    def fill(k, _):
        ones[k] = jnp.ones((16,), F32)
        return 0

    lax.fori_loop(0, CH, fill, 0)

    def zfill(k, _):
        zb[k] = jnp.zeros((16,), F32)
        return 0

    lax.fori_loop(0, NB, zfill, 0)
    for kk in range(NNB):
        pltpu.sync_copy(zb, degs_sp.at[pl.ds(row0 + kk * NB, NB)])
        pltpu.sync_copy(zb, degd_sp.at[pl.ds(row0 + kk * NB, NB)])
    plsc.subcore_barrier()

    def gbody(g, _):
        pltpu.sync_copy(srcs.at[wid, pl.ds(g * GC, GC)], sidx)
        pltpu.sync_copy(dsts.at[wid, pl.ds(g * GC, GC)], didx)

        def ebody(j, _):
            pltpu.sync_copy(ones, degs_sp.at[sidx.at[j]], add=True)
            pltpu.sync_copy(ones, degd_sp.at[didx.at[j]], add=True)
            return 0

        lax.fori_loop(0, GC, ebody, 0)
        return 0

    lax.fori_loop(0, NG, gbody, 0)
    plsc.subcore_barrier()
    pltpu.sync_copy(degs_sp.at[pl.ds(row0, RT)], out.at[c, 0, pl.ds(row0, RT)])
    pltpu.sync_copy(degd_sp.at[pl.ds(row0, RT)], out.at[c, 1, pl.ds(row0, RT)])


_deg_kernel = functools.partial(
    pl.kernel,
    out_type=jax.ShapeDtypeStruct((NC, 2, NPAD, 16), F32),
    mesh=_mesh,
    compiler_params=_sc_params,
    scratch_types=[
        pltpu.VMEM((GC, CH), jnp.int32),
        pltpu.VMEM((GC, CH), jnp.int32),
        pltpu.VMEM((CH, 16), F32),
        pltpu.VMEM((NB, 16), F32),
        pltpu.VMEM_SHARED((NPAD, 16), F32),
        pltpu.VMEM_SHARED((NPAD, 16), F32),
    ],
)(_deg_body)


def _comb_body(d_ref, z_ref, nd_ref, ns_ref, g0_ref, a0_ref):
    ds = d_ref[0, 0] + d_ref[1, 0]
    dd = d_ref[0, 1] + d_ref[1, 1]
    z = z_ref[...]
    rows = lax.broadcasted_iota(jnp.int32, (NPAD, 16), 0)
    valid = rows < N
    ns_ref[...] = jnp.where(valid, jnp.power(jnp.maximum(ds, 1.0), -0.5), 0.0)
    ndv = jnp.power(jnp.maximum(dd, 1.0), -0.5)
    nd_ref[...] = jnp.where(valid, (1.0 - ALPHA) * ndv, 0.0)
    tail = jnp.zeros((NPAD - N, 16), F32)
    g0_ref[:N] = ALPHA * z
    g0_ref[N:] = tail
    a0_ref[:N] = z / ndv[:N]
    a0_ref[N:] = tail


def _combine(degs, z0):
    shp = jax.ShapeDtypeStruct((NPAD, 16), F32)
    return pl.pallas_call(
        _comb_body,
        out_shape=(shp, shp, shp, shp),
    )(degs, z0)


def _prop_body(a0, a1, nd, ns, g0, srcs, dsts, out,
               sidx, didx, ba0, ba1, bnd, bns, bg0, bu, bz, gbuf, u_sp,
               acc_sp, sem):
    c = lax.axis_index("c")
    s = lax.axis_index("s")
    wid = c * NS + s
    row0 = s * RT

    def zfill(k, _):
        bz[k] = jnp.zeros((16,), F32)
        return 0

    lax.fori_loop(0, NB, zfill, 0)

    for kk in range(NNB):
        r0 = row0 + kk * NB
        pltpu.sync_copy(a0.at[pl.ds(r0, NB)], ba0)
        pltpu.sync_copy(a1.at[pl.ds(r0, NB)], ba1)
        pltpu.sync_copy(nd.at[pl.ds(r0, NB)], bnd)
        pltpu.sync_copy(ns.at[pl.ds(r0, NB)], bns)
        pltpu.sync_copy(g0.at[pl.ds(r0, NB)], bg0)

        def nbody(k, _):
            feat = bnd[k] * (ba0[k] + ba1[k]) + bg0[k]
            bu[k] = bns[k] * feat
            return 0

        lax.fori_loop(0, NB, nbody, 0)
        pltpu.sync_copy(bu, u_sp.at[pl.ds(r0, NB)])
        pltpu.sync_copy(bz, acc_sp.at[pl.ds(r0, NB)])

    plsc.subcore_barrier()

    def gbody(g, _):
        pltpu.sync_copy(srcs.at[wid, pl.ds(g * GC, GC)], sidx)
        pltpu.sync_copy(dsts.at[wid, pl.ds(g * GC, GC)], didx)

        def ebody(j, _):
            pltpu.async_copy(u_sp.at[sidx.at[j]], gbuf, sem).wait()
            pltpu.sync_copy(gbuf, acc_sp.at[didx.at[j]], add=True)
            return 0

        lax.fori_loop(0, GC, ebody, 0)
        return 0

    lax.fori_loop(0, NG, gbody, 0)
    plsc.subcore_barrier()
    pltpu.sync_copy(acc_sp.at[pl.ds(row0, RT)], out.at[c, pl.ds(row0, RT)])


_prop_kernel = functools.partial(
    pl.kernel,
    out_type=jax.ShapeDtypeStruct((NC, NPAD, 16), F32),
    mesh=_mesh,
    compiler_params=_sc_params,
    scratch_types=[
        pltpu.VMEM((GC, CH), jnp.int32),
        pltpu.VMEM((GC, CH), jnp.int32),
        pltpu.VMEM((NB, 16), F32),
        pltpu.VMEM((NB, 16), F32),
        pltpu.VMEM((NB, 16), F32),
        pltpu.VMEM((NB, 16), F32),
        pltpu.VMEM((NB, 16), F32),
        pltpu.VMEM((NB, 16), F32),
        pltpu.VMEM((NB, 16), F32),
        pltpu.VMEM((CH, 16), F32),
        pltpu.VMEM_SHARED((NPAD, 16), F32),
        pltpu.VMEM_SHARED((NPAD, 16), F32),
        pltpu.SemaphoreType.DMA,
    ],
)(_prop_body)


def _final_body(a0_ref, a1_ref, nd_ref, g0_ref, o_ref):
    o_ref[...] = (nd_ref[:N] * (a0_ref[:N] + a1_ref[:N]) + g0_ref[:N])


def _final(a0, a1, nd, g0):
    return pl.pallas_call(
        _final_body,
        out_shape=jax.ShapeDtypeStruct((N, 16), F32),
    )(a0, a1, nd, g0)


def kernel(in_feat, edge_index, W1, b1, W2, b2):
    edge_index = edge_index.astype(jnp.int32)
    srcs = edge_index[0].reshape(NW, NCH, CH)
    dsts = edge_index[1].reshape(NW, NCH, CH)

    z0 = _mlp(in_feat, W1, b1, W2, b2)
    degs = _deg_kernel(srcs, dsts)
    nd, ns, g0, a0 = _combine(degs, z0)
    a1 = jnp.zeros((NPAD, 16), F32)
    for _ in range(K):
        acc = _prop_kernel(a0, a1, nd, ns, g0, srcs, dsts)
        a0, a1 = acc[0], acc[1]
    return _final(a0, a1, nd, g0)[:N]


# speedup vs baseline: 18.8919x; 1.1648x over previous
import functools

import jax
import jax.numpy as jnp
from jax import lax
from jax.experimental import pallas as pl
from jax.experimental.pallas import tpu as pltpu
from jax.experimental.pallas import tpu_sc as plsc

N = 10000
E = 320000
IN_FEATS = 128
H_FEATS = 64
NUM_CLASSES = 16
K = 10
ALPHA = 0.1

NC = 2
NS = 16
NW = NC * NS
NPAD = 10240
RT = NPAD // NS
NB = 128
NNB = RT // NB
CH = 625
EW = E // NW
NCH = EW // CH
GC = 8
NG = NCH // GC
F32 = jnp.float32

_mesh = plsc.VectorSubcoreMesh(
    core_axis_name="c", subcore_axis_name="s", num_cores=NC, num_subcores=NS
)
_sc_params = pltpu.CompilerParams(use_tc_tiling_on_sc=False)


def _mlp_body(x_ref, w1_ref, b1_ref, w2_ref, b2_ref, o_ref):
    x = x_ref[...]
    h = lax.dot_general(x, w1_ref[...], (((1,), (1,)), ((), ())),
                        preferred_element_type=F32)
    h = jnp.maximum(h + b1_ref[...], 0.0)
    o = lax.dot_general(h, w2_ref[...], (((1,), (1,)), ((), ())),
                        preferred_element_type=F32)
    o_ref[...] = o + b2_ref[...]


def _mlp(in_feat, W1, b1, W2, b2):
    blk = 2000
    grid = (N // blk,)
    return pl.pallas_call(
        _mlp_body,
        grid=grid,
        in_specs=[
            pl.BlockSpec((blk, IN_FEATS), lambda i: (i, 0)),
            pl.BlockSpec((H_FEATS, IN_FEATS), lambda i: (0, 0)),
            pl.BlockSpec((1, H_FEATS), lambda i: (0, 0)),
            pl.BlockSpec((NUM_CLASSES, H_FEATS), lambda i: (0, 0)),
            pl.BlockSpec((1, NUM_CLASSES), lambda i: (0, 0)),
        ],
        out_specs=pl.BlockSpec((blk, NUM_CLASSES), lambda i: (i, 0)),
        out_shape=jax.ShapeDtypeStruct((N, NUM_CLASSES), F32),
    )(in_feat, W1, b1.reshape(1, H_FEATS), W2, b2.reshape(1, NUM_CLASSES))


def _deg_body(srcs, dsts, out, sidx, didx, ones, zb, degs_sp, degd_sp):
    c = lax.axis_index("c")
    s = lax.axis_index("s")
    wid = c * NS + s
    row0 = s * RT

    def fill(k, _):
        ones[k] = jnp.ones((16,), F32)
        return 0

    lax.fori_loop(0, CH, fill, 0)

    def zfill(k, _):
        zb[k] = jnp.zeros((16,), F32)
        return 0

    lax.fori_loop(0, NB, zfill, 0)
    for kk in range(NNB):
        pltpu.sync_copy(zb, degs_sp.at[pl.ds(row0 + kk * NB, NB)])
        pltpu.sync_copy(zb, degd_sp.at[pl.ds(row0 + kk * NB, NB)])
    plsc.subcore_barrier()

    def gbody(g, _):
        pltpu.sync_copy(srcs.at[wid, pl.ds(g * GC, GC)], sidx)
        pltpu.sync_copy(dsts.at[wid, pl.ds(g * GC, GC)], didx)

        def ebody(j, _):
            pltpu.sync_copy(ones, degs_sp.at[sidx.at[j]], add=True)
            pltpu.sync_copy(ones, degd_sp.at[didx.at[j]], add=True)
            return 0

        lax.fori_loop(0, GC, ebody, 0)
        return 0

    lax.fori_loop(0, NG, gbody, 0)
    plsc.subcore_barrier()
    pltpu.sync_copy(degs_sp.at[pl.ds(row0, RT)], out.at[c, 0, pl.ds(row0, RT)])
    pltpu.sync_copy(degd_sp.at[pl.ds(row0, RT)], out.at[c, 1, pl.ds(row0, RT)])


_deg_kernel = functools.partial(
    pl.kernel,
    out_type=jax.ShapeDtypeStruct((NC, 2, NPAD, 16), F32),
    mesh=_mesh,
    compiler_params=_sc_params,
    scratch_types=[
        pltpu.VMEM((GC, CH), jnp.int32),
        pltpu.VMEM((GC, CH), jnp.int32),
        pltpu.VMEM((CH, 16), F32),
        pltpu.VMEM((NB, 16), F32),
        pltpu.VMEM_SHARED((NPAD, 16), F32),
        pltpu.VMEM_SHARED((NPAD, 16), F32),
    ],
)(_deg_body)


def _comb_body(d_ref, z_ref, nd_ref, ns_ref, g0_ref, a0_ref):
    ds = d_ref[0, 0] + d_ref[1, 0]
    dd = d_ref[0, 1] + d_ref[1, 1]
    z = z_ref[...]
    rows = lax.broadcasted_iota(jnp.int32, (NPAD, 16), 0)
    valid = rows < N
    ns_ref[...] = jnp.where(valid, jnp.power(jnp.maximum(ds, 1.0), -0.5), 0.0)
    ndv = jnp.power(jnp.maximum(dd, 1.0), -0.5)
    nd_ref[...] = jnp.where(valid, (1.0 - ALPHA) * ndv, 0.0)
    tail = jnp.zeros((NPAD - N, 16), F32)
    g0_ref[:N] = ALPHA * z
    g0_ref[N:] = tail
    a0_ref[:N] = z / ndv[:N]
    a0_ref[N:] = tail


def _combine(degs, z0):
    shp = jax.ShapeDtypeStruct((NPAD, 16), F32)
    return pl.pallas_call(
        _comb_body,
        out_shape=(shp, shp, shp, shp),
    )(degs, z0)


def _prop_body(a0, a1, nd, ns, g0, srcs, dsts, out,
               sidx, didx, ba0, ba1, bnd, bns, bg0, bu, bz, gbuf, u_sp,
               acc_sp, sem):
    c = lax.axis_index("c")
    s = lax.axis_index("s")
    wid = c * NS + s
    row0 = s * RT

    def zfill(k, _):
        bz[k] = jnp.zeros((16,), F32)
        return 0

    lax.fori_loop(0, NB, zfill, 0)

    for kk in range(NNB):
        r0 = row0 + kk * NB
        pltpu.sync_copy(a0.at[pl.ds(r0, NB)], ba0)
        pltpu.sync_copy(a1.at[pl.ds(r0, NB)], ba1)
        pltpu.sync_copy(nd.at[pl.ds(r0, NB)], bnd)
        pltpu.sync_copy(ns.at[pl.ds(r0, NB)], bns)
        pltpu.sync_copy(g0.at[pl.ds(r0, NB)], bg0)

        def nbody(k, _):
            feat = bnd[k] * (ba0[k] + ba1[k]) + bg0[k]
            bu[k] = bns[k] * feat
            return 0

        lax.fori_loop(0, NB, nbody, 0)
        pltpu.sync_copy(bu, u_sp.at[pl.ds(r0, NB)])
        pltpu.sync_copy(bz, acc_sp.at[pl.ds(r0, NB)])

    plsc.subcore_barrier()

    def gbody(g, _):
        pltpu.sync_copy(srcs.at[wid, pl.ds(g * GC, GC)], sidx)
        pltpu.sync_copy(dsts.at[wid, pl.ds(g * GC, GC)], didx)

        def ebody(j, _):
            pltpu.async_copy(u_sp.at[sidx.at[j]], gbuf, sem).wait()
            pltpu.sync_copy(gbuf, acc_sp.at[didx.at[j]], add=True)
            return 0

        lax.fori_loop(0, GC, ebody, 0)
        return 0

    lax.fori_loop(0, NG, gbody, 0)
    plsc.subcore_barrier()
    pltpu.sync_copy(acc_sp.at[pl.ds(row0, RT)], out.at[c, pl.ds(row0, RT)])


_prop_kernel = functools.partial(
    pl.kernel,
    out_type=jax.ShapeDtypeStruct((NC, NPAD, 16), F32),
    mesh=_mesh,
    compiler_params=_sc_params,
    scratch_types=[
        pltpu.VMEM((GC, CH), jnp.int32),
        pltpu.VMEM((GC, CH), jnp.int32),
        pltpu.VMEM((NB, 16), F32),
        pltpu.VMEM((NB, 16), F32),
        pltpu.VMEM((NB, 16), F32),
        pltpu.VMEM((NB, 16), F32),
        pltpu.VMEM((NB, 16), F32),
        pltpu.VMEM((NB, 16), F32),
        pltpu.VMEM((NB, 16), F32),
        pltpu.VMEM((CH, 16), F32),
        pltpu.VMEM_SHARED((NPAD, 16), F32),
        pltpu.VMEM_SHARED((NPAD, 16), F32),
        pltpu.SemaphoreType.DMA,
    ],
)(_prop_body)


def _final_body(a0_ref, a1_ref, nd_ref, g0_ref, o_ref):
    o_ref[...] = (nd_ref[:N] * (a0_ref[:N] + a1_ref[:N]) + g0_ref[:N])


def _final(a0, a1, nd, g0):
    return pl.pallas_call(
        _final_body,
        out_shape=jax.ShapeDtypeStruct((N, 16), F32),
    )(a0, a1, nd, g0)


def kernel(in_feat, edge_index, W1, b1, W2, b2):
    edge_index = edge_index.astype(jnp.int32)
    srcs = edge_index[0].reshape(NW, NCH, CH)
    dsts = edge_index[1].reshape(NW, NCH, CH)

    z0 = _mlp(in_feat, W1, b1, W2, b2)
    degs = _deg_kernel(srcs, dsts)
    nd, ns, g0, a0 = _combine(degs, z0)
    a1 = jnp.zeros((NPAD, 16), F32)
    for _ in range(K):
        acc = _prop_kernel(a0, a1, nd, ns, g0, srcs, dsts)
        a0, a1 = acc[0], acc[1]
    return _final(a0, a1, nd, g0)[:N]


# speedup vs baseline: 21.1805x; 1.1211x over previous
import functools

import jax
import jax.numpy as jnp
from jax import lax
from jax.experimental import pallas as pl
from jax.experimental.pallas import tpu as pltpu
from jax.experimental.pallas import tpu_sc as plsc

N = 10000
E = 320000
IN_FEATS = 128
H_FEATS = 64
NUM_CLASSES = 16
K = 10
ALPHA = 0.1

NC = 2
NS = 16
NW = NC * NS
NPAD = 10240
RT = NPAD // NS
NB = 128
NNB = RT // NB
EW = E // NW
PCH = 1250
PNCH = EW // PCH
CH = 125
NCH = EW // CH
GC = 8
NG = NCH // GC
F32 = jnp.float32

_mesh = plsc.VectorSubcoreMesh(
    core_axis_name="c", subcore_axis_name="s", num_cores=NC, num_subcores=NS
)
_sc_params = pltpu.CompilerParams(use_tc_tiling_on_sc=False)


def _mlp_body(x_ref, w1_ref, b1_ref, w2_ref, b2_ref, o_ref):
    x = x_ref[...]
    h = lax.dot_general(x, w1_ref[...], (((1,), (1,)), ((), ())),
                        preferred_element_type=F32)
    h = jnp.maximum(h + b1_ref[...], 0.0)
    o = lax.dot_general(h, w2_ref[...], (((1,), (1,)), ((), ())),
                        preferred_element_type=F32)
    o_ref[...] = o + b2_ref[...]


def _mlp(in_feat, W1, b1, W2, b2):
    blk = 2000
    grid = (N // blk,)
    return pl.pallas_call(
        _mlp_body,
        grid=grid,
        in_specs=[
            pl.BlockSpec((blk, IN_FEATS), lambda i: (i, 0)),
            pl.BlockSpec((H_FEATS, IN_FEATS), lambda i: (0, 0)),
            pl.BlockSpec((1, H_FEATS), lambda i: (0, 0)),
            pl.BlockSpec((NUM_CLASSES, H_FEATS), lambda i: (0, 0)),
            pl.BlockSpec((1, NUM_CLASSES), lambda i: (0, 0)),
        ],
        out_specs=pl.BlockSpec((blk, NUM_CLASSES), lambda i: (i, 0)),
        out_shape=jax.ShapeDtypeStruct((N, NUM_CLASSES), F32),
    )(in_feat, W1, b1.reshape(1, H_FEATS), W2, b2.reshape(1, NUM_CLASSES))


def _deg_body(srcs, dsts, out, sidx, didx, ones, zb, degs_sp, degd_sp):
    c = lax.axis_index("c")
    s = lax.axis_index("s")
    wid = c * NS + s
    row0 = s * RT

    def fill(k, _):
        ones[k] = jnp.ones((16,), F32)
        return 0

    lax.fori_loop(0, CH, fill, 0)

    def zfill(k, _):
        zb[k] = jnp.zeros((16,), F32)
        return 0

    lax.fori_loop(0, NB, zfill, 0)
    for kk in range(NNB):
        pltpu.sync_copy(zb, degs_sp.at[pl.ds(row0 + kk * NB, NB)])
        pltpu.sync_copy(zb, degd_sp.at[pl.ds(row0 + kk * NB, NB)])
    plsc.subcore_barrier()

    def gbody(g, _):
        pltpu.sync_copy(srcs.at[wid, pl.ds(g * GC, GC)], sidx)
        pltpu.sync_copy(dsts.at[wid, pl.ds(g * GC, GC)], didx)

        def ebody(j, _):
            pltpu.sync_copy(ones, degs_sp.at[sidx.at[j]], add=True)
            pltpu.sync_copy(ones, degd_sp.at[didx.at[j]], add=True)
            return 0

        lax.fori_loop(0, GC, ebody, 0)
        return 0

    lax.fori_loop(0, NG, gbody, 0)
    plsc.subcore_barrier()
    pltpu.sync_copy(degs_sp.at[pl.ds(row0, RT)], out.at[c, 0, pl.ds(row0, RT)])
    pltpu.sync_copy(degd_sp.at[pl.ds(row0, RT)], out.at[c, 1, pl.ds(row0, RT)])


_deg_kernel = functools.partial(
    pl.kernel,
    out_type=jax.ShapeDtypeStruct((NC, 2, NPAD, 16), F32),
    mesh=_mesh,
    compiler_params=_sc_params,
    scratch_types=[
        pltpu.VMEM((GC, CH), jnp.int32),
        pltpu.VMEM((GC, CH), jnp.int32),
        pltpu.VMEM((CH, 16), F32),
        pltpu.VMEM((NB, 16), F32),
        pltpu.VMEM_SHARED((NPAD, 16), F32),
        pltpu.VMEM_SHARED((NPAD, 16), F32),
    ],
)(_deg_body)


def _comb_body(d_ref, z_ref, nd_ref, ns_ref, g0_ref, a0_ref):
    ds = d_ref[0, 0] + d_ref[1, 0]
    dd = d_ref[0, 1] + d_ref[1, 1]
    z = z_ref[...]
    rows = lax.broadcasted_iota(jnp.int32, (NPAD, 16), 0)
    valid = rows < N
    ns_ref[...] = jnp.where(valid, jnp.power(jnp.maximum(ds, 1.0), -0.5), 0.0)
    ndv = jnp.power(jnp.maximum(dd, 1.0), -0.5)
    nd_ref[...] = jnp.where(valid, (1.0 - ALPHA) * ndv, 0.0)
    tail = jnp.zeros((NPAD - N, 16), F32)
    g0_ref[:N] = ALPHA * z
    g0_ref[N:] = tail
    a0_ref[:N] = z / ndv[:N]
    a0_ref[N:] = tail


def _combine(degs, z0):
    shp = jax.ShapeDtypeStruct((NPAD, 16), F32)
    return pl.pallas_call(
        _comb_body,
        out_shape=(shp, shp, shp, shp),
    )(degs, z0)


def _prop_body(a0, a1, nd, ns, g0, srcs, dsts, out,
               sidx, didx, ba0, ba1, bnd, bns, bg0, bu, bz, gb0, gb1, u_sp,
               acc_sp, gs0, gs1, ss0, ss1):
    c = lax.axis_index("c")
    s = lax.axis_index("s")
    wid = c * NS + s
    row0 = s * RT

    def zfill(k, _):
        bz[k] = jnp.zeros((16,), F32)
        return 0

    lax.fori_loop(0, NB, zfill, 0)

    for kk in range(NNB):
        r0 = row0 + kk * NB
        pltpu.sync_copy(a0.at[pl.ds(r0, NB)], ba0)
        pltpu.sync_copy(a1.at[pl.ds(r0, NB)], ba1)
        pltpu.sync_copy(nd.at[pl.ds(r0, NB)], bnd)
        pltpu.sync_copy(ns.at[pl.ds(r0, NB)], bns)
        pltpu.sync_copy(g0.at[pl.ds(r0, NB)], bg0)

        def nbody(k, _):
            feat = bnd[k] * (ba0[k] + ba1[k]) + bg0[k]
            bu[k] = bns[k] * feat
            return 0

        lax.fori_loop(0, NB, nbody, 0)
        pltpu.sync_copy(bu, u_sp.at[pl.ds(r0, NB)])
        pltpu.sync_copy(bz, acc_sp.at[pl.ds(r0, NB)])

    pltpu.sync_copy(srcs.at[wid], sidx)
    pltpu.sync_copy(dsts.at[wid], didx)
    plsc.subcore_barrier()

    bufs = [gb0, gb1]
    gsems = [gs0, gs1]
    ssems = [ss0, ss1]
    gcp = [
        pltpu.async_copy(u_sp.at[sidx.at[0]], gb0, gs0),
        pltpu.async_copy(u_sp.at[sidx.at[1]], gb1, gs1),
    ]
    for j in range(PNCH):
        b = j & 1
        gcp[b].wait()
        scp = pltpu.async_copy(
            bufs[b], acc_sp.at[didx.at[j]], ssems[b], add=True)
        scp.wait()
        if j + 2 < PNCH:
            gcp[b] = pltpu.async_copy(
                u_sp.at[sidx.at[j + 2]], bufs[b], gsems[b])
    plsc.subcore_barrier()
    pltpu.sync_copy(acc_sp.at[pl.ds(row0, RT)], out.at[c, pl.ds(row0, RT)])


_prop_kernel = functools.partial(
    pl.kernel,
    out_type=jax.ShapeDtypeStruct((NC, NPAD, 16), F32),
    mesh=_mesh,
    compiler_params=_sc_params,
    scratch_types=[
        pltpu.VMEM((PNCH, PCH), jnp.int32),
        pltpu.VMEM((PNCH, PCH), jnp.int32),
        pltpu.VMEM((NB, 16), F32),
        pltpu.VMEM((NB, 16), F32),
        pltpu.VMEM((NB, 16), F32),
        pltpu.VMEM((NB, 16), F32),
        pltpu.VMEM((NB, 16), F32),
        pltpu.VMEM((NB, 16), F32),
        pltpu.VMEM((NB, 16), F32),
        pltpu.VMEM((PCH, 16), F32),
        pltpu.VMEM((PCH, 16), F32),
        pltpu.VMEM_SHARED((NPAD, 16), F32),
        pltpu.VMEM_SHARED((NPAD, 16), F32),
        pltpu.SemaphoreType.DMA,
        pltpu.SemaphoreType.DMA,
        pltpu.SemaphoreType.DMA,
        pltpu.SemaphoreType.DMA,
    ],
)(_prop_body)


def _final_body(a0_ref, a1_ref, nd_ref, g0_ref, o_ref):
    o_ref[...] = (nd_ref[:N] * (a0_ref[:N] + a1_ref[:N]) + g0_ref[:N])


def _final(a0, a1, nd, g0):
    return pl.pallas_call(
        _final_body,
        out_shape=jax.ShapeDtypeStruct((N, 16), F32),
    )(a0, a1, nd, g0)


def kernel(in_feat, edge_index, W1, b1, W2, b2):
    edge_index = edge_index.astype(jnp.int32)
    srcs_p = edge_index[0].reshape(NW, PNCH, PCH)
    dsts_p = edge_index[1].reshape(NW, PNCH, PCH)
    srcs = edge_index[0].reshape(NW, NCH, CH)
    dsts = edge_index[1].reshape(NW, NCH, CH)

    z0 = _mlp(in_feat, W1, b1, W2, b2)
    degs = _deg_kernel(srcs, dsts)
    nd, ns, g0, a0 = _combine(degs, z0)
    a1 = jnp.zeros((NPAD, 16), F32)
    for _ in range(K):
        acc = _prop_kernel(a0, a1, nd, ns, g0, srcs_p, dsts_p)
        a0, a1 = acc[0], acc[1]
    return _final(a0, a1, nd, g0)[:N]


# speedup vs baseline: 24.0552x; 1.1357x over previous
import functools

import jax
import jax.numpy as jnp
from jax import lax
from jax.experimental import pallas as pl
from jax.experimental.pallas import tpu as pltpu
from jax.experimental.pallas import tpu_sc as plsc

N = 10000
E = 320000
IN_FEATS = 128
H_FEATS = 64
NUM_CLASSES = 16
K = 10
ALPHA = 0.1

NC = 2
NS = 16
NW = NC * NS
NPAD = 10240
RT = NPAD // NS
NB = 128
NNB = RT // NB
EW = E // NW
PCH = 1250
PNCH = EW // PCH
CH = 125
NCH = EW // CH
GC = 8
NG = NCH // GC
FNCH = E // NS // PCH
FGC = 4
FNG = FNCH // FGC
NH = NPAD // 2
RTF = NH // NS
F32 = jnp.float32

_mesh = plsc.VectorSubcoreMesh(
    core_axis_name="c", subcore_axis_name="s", num_cores=NC, num_subcores=NS
)
_sc_params = pltpu.CompilerParams(use_tc_tiling_on_sc=False)


def _mlp_body(x_ref, w1_ref, b1_ref, w2_ref, b2_ref, o_ref):
    x = x_ref[...]
    h = lax.dot_general(x, w1_ref[...], (((1,), (1,)), ((), ())),
                        preferred_element_type=F32)
    h = jnp.maximum(h + b1_ref[...], 0.0)
    o = lax.dot_general(h, w2_ref[...], (((1,), (1,)), ((), ())),
                        preferred_element_type=F32)
    o_ref[...] = o + b2_ref[...]


def _mlp(in_feat, W1, b1, W2, b2):
    blk = 2000
    grid = (N // blk,)
    return pl.pallas_call(
        _mlp_body,
        grid=grid,
        in_specs=[
            pl.BlockSpec((blk, IN_FEATS), lambda i: (i, 0)),
            pl.BlockSpec((H_FEATS, IN_FEATS), lambda i: (0, 0)),
            pl.BlockSpec((1, H_FEATS), lambda i: (0, 0)),
            pl.BlockSpec((NUM_CLASSES, H_FEATS), lambda i: (0, 0)),
            pl.BlockSpec((1, NUM_CLASSES), lambda i: (0, 0)),
        ],
        out_specs=pl.BlockSpec((blk, NUM_CLASSES), lambda i: (i, 0)),
        out_shape=jax.ShapeDtypeStruct((N, NUM_CLASSES), F32),
    )(in_feat, W1, b1.reshape(1, H_FEATS), W2, b2.reshape(1, NUM_CLASSES))


def _deg_body(srcs, dsts, out, sidx, didx, ones, zb, degs_sp, degd_sp):
    c = lax.axis_index("c")
    s = lax.axis_index("s")
    wid = c * NS + s
    row0 = s * RT

    def fill(k, _):
        ones[k] = jnp.ones((16,), F32)
        return 0

    lax.fori_loop(0, CH, fill, 0)

    def zfill(k, _):
        zb[k] = jnp.zeros((16,), F32)
        return 0

    lax.fori_loop(0, NB, zfill, 0)
    for kk in range(NNB):
        pltpu.sync_copy(zb, degs_sp.at[pl.ds(row0 + kk * NB, NB)])
        pltpu.sync_copy(zb, degd_sp.at[pl.ds(row0 + kk * NB, NB)])
    plsc.subcore_barrier()

    def gbody(g, _):
        pltpu.sync_copy(srcs.at[wid, pl.ds(g * GC, GC)], sidx)
        pltpu.sync_copy(dsts.at[wid, pl.ds(g * GC, GC)], didx)

        def ebody(j, _):
            pltpu.sync_copy(ones, degs_sp.at[sidx.at[j]], add=True)
            pltpu.sync_copy(ones, degd_sp.at[didx.at[j]], add=True)
            return 0

        lax.fori_loop(0, GC, ebody, 0)
        return 0

    lax.fori_loop(0, NG, gbody, 0)
    plsc.subcore_barrier()
    pltpu.sync_copy(degs_sp.at[pl.ds(row0, RT)], out.at[c, 0, pl.ds(row0, RT)])
    pltpu.sync_copy(degd_sp.at[pl.ds(row0, RT)], out.at[c, 1, pl.ds(row0, RT)])


_deg_kernel = functools.partial(
    pl.kernel,
    out_type=jax.ShapeDtypeStruct((NC, 2, NPAD, 16), F32),
    mesh=_mesh,
    compiler_params=_sc_params,
    scratch_types=[
        pltpu.VMEM((GC, CH), jnp.int32),
        pltpu.VMEM((GC, CH), jnp.int32),
        pltpu.VMEM((CH, 16), F32),
        pltpu.VMEM((NB, 16), F32),
        pltpu.VMEM_SHARED((NPAD, 16), F32),
        pltpu.VMEM_SHARED((NPAD, 16), F32),
    ],
)(_deg_body)


def _comb_body(d_ref, z_ref, nd_ref, ns_ref, g0_ref, a0_ref):
    ds = d_ref[0, 0] + d_ref[1, 0]
    dd = d_ref[0, 1] + d_ref[1, 1]
    z = z_ref[...]
    rows = lax.broadcasted_iota(jnp.int32, (NPAD, 16), 0)
    valid = rows < N
    ns_ref[...] = jnp.where(valid, jnp.power(jnp.maximum(ds, 1.0), -0.5), 0.0)
    ndv = jnp.power(jnp.maximum(dd, 1.0), -0.5)
    nd_ref[...] = jnp.where(valid, (1.0 - ALPHA) * ndv, 0.0)
    tail = jnp.zeros((NPAD - N, 16), F32)
    g0_ref[:N] = ALPHA * z
    g0_ref[N:] = tail
    a0_ref[:N] = z / ndv[:N]
    a0_ref[N:] = tail


def _combine(degs, z0):
    shp = jax.ShapeDtypeStruct((NPAD, 16), F32)
    return pl.pallas_call(
        _comb_body,
        out_shape=(shp, shp, shp, shp),
    )(degs, z0)


def _fprop_body(a0i, nd, ns, g0, srcs, dsts, out,
                sidx, didx, ba, bnd, bns, bg0, bu, bz, gb0, gb1, u_sp,
                acc_sp, gs0, gs1, ss0, ss1):
    s = lax.axis_index("s")
    c = lax.axis_index("c")
    row0 = s * RT

    def zfill(k, _):
        bz[k] = jnp.zeros((16,), F32)
        return 0

    lax.fori_loop(0, NB, zfill, 0)

    def row_phase(first):
        for kk in range(NNB):
            r0 = row0 + kk * NB
            if first:
                pltpu.sync_copy(a0i.at[pl.ds(r0, NB)], ba)
            else:
                pltpu.sync_copy(acc_sp.at[pl.ds(r0, NB)], ba)
            pltpu.sync_copy(nd.at[pl.ds(r0, NB)], bnd)
            pltpu.sync_copy(ns.at[pl.ds(r0, NB)], bns)
            pltpu.sync_copy(g0.at[pl.ds(r0, NB)], bg0)

            def nbody(k, _):
                feat = bnd[k] * ba[k] + bg0[k]
                bu[k] = bns[k] * feat
                return 0

            lax.fori_loop(0, NB, nbody, 0)
            pltpu.sync_copy(bu, u_sp.at[pl.ds(r0, NB)])
            pltpu.sync_copy(bz, acc_sp.at[pl.ds(r0, NB)])

    def edge_phase():
        bufs = [gb0, gb1]
        gsems = [gs0, gs1]
        ssems = [ss0, ss1]
        for g in range(FNG):
            pltpu.sync_copy(srcs.at[s, pl.ds(g * FGC, FGC)], sidx)
            pltpu.sync_copy(dsts.at[s, pl.ds(g * FGC, FGC)], didx)
            gcp = [
                pltpu.async_copy(u_sp.at[sidx.at[0]], gb0, gs0),
                pltpu.async_copy(u_sp.at[sidx.at[1]], gb1, gs1),
            ]
            for j in range(FGC):
                b = j & 1
                gcp[b].wait()
                scp = pltpu.async_copy(
                    bufs[b], acc_sp.at[didx.at[j]], ssems[b], add=True)
                scp.wait()
                if j + 2 < FGC:
                    gcp[b] = pltpu.async_copy(
                        u_sp.at[sidx.at[j + 2]], bufs[b], gsems[b])

    row_phase(first=True)
    plsc.subcore_barrier()
    for r in range(K):
        edge_phase()
        plsc.subcore_barrier()
        if r < K - 1:
            row_phase(first=False)
            plsc.subcore_barrier()

    f0 = c * NH + s * RTF
    for off, sz in ((0, NB), (NB, NB), (2 * NB, RTF - 2 * NB)):
        rf = f0 + off
        pltpu.sync_copy(acc_sp.at[pl.ds(rf, sz)], ba.at[pl.ds(0, sz)])
        pltpu.sync_copy(nd.at[pl.ds(rf, sz)], bnd.at[pl.ds(0, sz)])
        pltpu.sync_copy(g0.at[pl.ds(rf, sz)], bg0.at[pl.ds(0, sz)])

        def fbody(k, _):
            bu[k] = bnd[k] * ba[k] + bg0[k]
            return 0

        lax.fori_loop(0, sz, fbody, 0)
        pltpu.sync_copy(bu.at[pl.ds(0, sz)], out.at[pl.ds(rf, sz)])


_fprop_kernel = functools.partial(
    pl.kernel,
    out_type=jax.ShapeDtypeStruct((NPAD, 16), F32),
    mesh=_mesh,
    compiler_params=_sc_params,
    scratch_types=[
        pltpu.VMEM((FGC, PCH), jnp.int32),
        pltpu.VMEM((FGC, PCH), jnp.int32),
        pltpu.VMEM((NB, 16), F32),
        pltpu.VMEM((NB, 16), F32),
        pltpu.VMEM((NB, 16), F32),
        pltpu.VMEM((NB, 16), F32),
        pltpu.VMEM((NB, 16), F32),
        pltpu.VMEM((NB, 16), F32),
        pltpu.VMEM((PCH, 16), F32),
        pltpu.VMEM((PCH, 16), F32),
        pltpu.VMEM_SHARED((NPAD, 16), F32),
        pltpu.VMEM_SHARED((NPAD, 16), F32),
        pltpu.SemaphoreType.DMA,
        pltpu.SemaphoreType.DMA,
        pltpu.SemaphoreType.DMA,
        pltpu.SemaphoreType.DMA,
    ],
)(_fprop_body)


def _final_body(f_ref, o_ref):
    o_ref[...] = f_ref[:N]


def _final(feat):
    return pl.pallas_call(
        _final_body,
        out_shape=jax.ShapeDtypeStruct((N, 16), F32),
    )(feat)


def kernel(in_feat, edge_index, W1, b1, W2, b2):
    edge_index = edge_index.astype(jnp.int32)
    srcs_f = edge_index[0].reshape(NS, FNCH, PCH)
    dsts_f = edge_index[1].reshape(NS, FNCH, PCH)
    srcs = edge_index[0].reshape(NW, NCH, CH)
    dsts = edge_index[1].reshape(NW, NCH, CH)

    z0 = _mlp(in_feat, W1, b1, W2, b2)
    degs = _deg_kernel(srcs, dsts)
    nd, ns, g0, a0 = _combine(degs, z0)
    feat = _fprop_kernel(a0, nd, ns, g0, srcs_f, dsts_f)
    return _final(feat)


# speedup vs baseline: 28.0120x; 1.1645x over previous
import functools

import jax
import jax.numpy as jnp
from jax import lax
from jax.experimental import pallas as pl
from jax.experimental.pallas import tpu as pltpu
from jax.experimental.pallas import tpu_sc as plsc

N = 10000
E = 320000
IN_FEATS = 128
H_FEATS = 64
NUM_CLASSES = 16
K = 10
ALPHA = 0.1

NC = 2
NS = 16
NW = NC * NS
NPAD = 10240
RT = NPAD // NS
NB = 128
NNB = RT // NB
EW = E // NW
PCH = 625
PNCH = EW // PCH
CH = 125
NCH = EW // CH
GC = 8
NG = NCH // GC
FNCH = E // NS // PCH
NH = NPAD // 2
RTF = NH // NS
F32 = jnp.float32

_mesh = plsc.VectorSubcoreMesh(
    core_axis_name="c", subcore_axis_name="s", num_cores=NC, num_subcores=NS
)
_sc_params = pltpu.CompilerParams(use_tc_tiling_on_sc=False)


def _mlp_body(x_ref, w1_ref, b1_ref, w2_ref, b2_ref, o_ref):
    x = x_ref[...]
    h = lax.dot_general(x, w1_ref[...], (((1,), (1,)), ((), ())),
                        preferred_element_type=F32)
    h = jnp.maximum(h + b1_ref[...], 0.0)
    o = lax.dot_general(h, w2_ref[...], (((1,), (1,)), ((), ())),
                        preferred_element_type=F32)
    o_ref[...] = o + b2_ref[...]


def _mlp(in_feat, W1, b1, W2, b2):
    blk = 2000
    grid = (N // blk,)
    return pl.pallas_call(
        _mlp_body,
        grid=grid,
        in_specs=[
            pl.BlockSpec((blk, IN_FEATS), lambda i: (i, 0)),
            pl.BlockSpec((H_FEATS, IN_FEATS), lambda i: (0, 0)),
            pl.BlockSpec((1, H_FEATS), lambda i: (0, 0)),
            pl.BlockSpec((NUM_CLASSES, H_FEATS), lambda i: (0, 0)),
            pl.BlockSpec((1, NUM_CLASSES), lambda i: (0, 0)),
        ],
        out_specs=pl.BlockSpec((blk, NUM_CLASSES), lambda i: (i, 0)),
        out_shape=jax.ShapeDtypeStruct((N, NUM_CLASSES), F32),
    )(in_feat, W1, b1.reshape(1, H_FEATS), W2, b2.reshape(1, NUM_CLASSES))


def _deg_body(srcs, dsts, out, sidx, didx, ones, zb, degs_sp, degd_sp):
    c = lax.axis_index("c")
    s = lax.axis_index("s")
    wid = c * NS + s
    row0 = s * RT

    def fill(k, _):
        ones[k] = jnp.ones((16,), F32)
        return 0

    lax.fori_loop(0, CH, fill, 0)

    def zfill(k, _):
        zb[k] = jnp.zeros((16,), F32)
        return 0

    lax.fori_loop(0, NB, zfill, 0)
    for kk in range(NNB):
        pltpu.sync_copy(zb, degs_sp.at[pl.ds(row0 + kk * NB, NB)])
        pltpu.sync_copy(zb, degd_sp.at[pl.ds(row0 + kk * NB, NB)])
    plsc.subcore_barrier()

    def gbody(g, _):
        pltpu.sync_copy(srcs.at[wid, pl.ds(g * GC, GC)], sidx)
        pltpu.sync_copy(dsts.at[wid, pl.ds(g * GC, GC)], didx)

        def ebody(j, _):
            pltpu.sync_copy(ones, degs_sp.at[sidx.at[j]], add=True)
            pltpu.sync_copy(ones, degd_sp.at[didx.at[j]], add=True)
            return 0

        lax.fori_loop(0, GC, ebody, 0)
        return 0

    lax.fori_loop(0, NG, gbody, 0)
    plsc.subcore_barrier()
    pltpu.sync_copy(degs_sp.at[pl.ds(row0, RT)], out.at[c, 0, pl.ds(row0, RT)])
    pltpu.sync_copy(degd_sp.at[pl.ds(row0, RT)], out.at[c, 1, pl.ds(row0, RT)])


_deg_kernel = functools.partial(
    pl.kernel,
    out_type=jax.ShapeDtypeStruct((NC, 2, NPAD, 16), F32),
    mesh=_mesh,
    compiler_params=_sc_params,
    scratch_types=[
        pltpu.VMEM((GC, CH), jnp.int32),
        pltpu.VMEM((GC, CH), jnp.int32),
        pltpu.VMEM((CH, 16), F32),
        pltpu.VMEM((NB, 16), F32),
        pltpu.VMEM_SHARED((NPAD, 16), F32),
        pltpu.VMEM_SHARED((NPAD, 16), F32),
    ],
)(_deg_body)


def _comb_body(d_ref, z_ref, nd_ref, ns_ref, g0_ref, a0_ref):
    ds = d_ref[0, 0] + d_ref[1, 0]
    dd = d_ref[0, 1] + d_ref[1, 1]
    z = z_ref[...]
    rows = lax.broadcasted_iota(jnp.int32, (NPAD, 16), 0)
    valid = rows < N
    ns_ref[...] = jnp.where(valid, jnp.power(jnp.maximum(ds, 1.0), -0.5), 0.0)
    ndv = jnp.power(jnp.maximum(dd, 1.0), -0.5)
    nd_ref[...] = jnp.where(valid, (1.0 - ALPHA) * ndv, 0.0)
    tail = jnp.zeros((NPAD - N, 16), F32)
    g0_ref[:N] = ALPHA * z
    g0_ref[N:] = tail
    a0_ref[:N] = z / ndv[:N]
    a0_ref[N:] = tail


def _combine(degs, z0):
    shp = jax.ShapeDtypeStruct((NPAD, 16), F32)
    return pl.pallas_call(
        _comb_body,
        out_shape=(shp, shp, shp, shp),
    )(degs, z0)


def _fprop_body(a0i, nd, ns, g0, srcs, dsts, out,
                sidx, didx, ba, bnd, bns, bg0, bu, bz, gb0, gb1, gb2, u_sp,
                acc_sp, gs0, gs1, gs2, ss0, ss1, ss2):
    s = lax.axis_index("s")
    c = lax.axis_index("c")
    row0 = s * RT

    def zfill(k, _):
        bz[k] = jnp.zeros((16,), F32)
        return 0

    lax.fori_loop(0, NB, zfill, 0)
    pltpu.sync_copy(srcs.at[s], sidx)
    pltpu.sync_copy(dsts.at[s], didx)

    def row_phase(first):
        for kk in range(NNB):
            r0 = row0 + kk * NB
            if first:
                pltpu.sync_copy(a0i.at[pl.ds(r0, NB)], ba)
            else:
                pltpu.sync_copy(acc_sp.at[pl.ds(r0, NB)], ba)
            pltpu.sync_copy(nd.at[pl.ds(r0, NB)], bnd)
            pltpu.sync_copy(ns.at[pl.ds(r0, NB)], bns)
            pltpu.sync_copy(g0.at[pl.ds(r0, NB)], bg0)

            def nbody(k, _):
                feat = bnd[k] * ba[k] + bg0[k]
                bu[k] = bns[k] * feat
                return 0

            lax.fori_loop(0, NB, nbody, 0)
            pltpu.sync_copy(bu, u_sp.at[pl.ds(r0, NB)])
            pltpu.sync_copy(bz, acc_sp.at[pl.ds(r0, NB)])

    def edge_phase():
        bufs = [gb0, gb1, gb2]
        gsems = [gs0, gs1, gs2]
        ssems = [ss0, ss1, ss2]
        gcp = [None, None, None]
        scp = [None, None, None]
        gcp[0] = pltpu.async_copy(u_sp.at[sidx.at[0]], gb0, gs0)
        gcp[1] = pltpu.async_copy(u_sp.at[sidx.at[1]], gb1, gs1)
        for j in range(FNCH):
            b = j % 3
            gcp[b].wait()
            scp[b] = pltpu.async_copy(
                bufs[b], acc_sp.at[didx.at[j]], ssems[b], add=True)
            if j + 2 < FNCH:
                bn = (j + 2) % 3
                if scp[bn] is not None:
                    scp[bn].wait()
                gcp[bn] = pltpu.async_copy(
                    u_sp.at[sidx.at[j + 2]], bufs[bn], gsems[bn])
        scp[(FNCH - 1) % 3].wait()
        scp[(FNCH - 2) % 3].wait()

    row_phase(first=True)
    plsc.subcore_barrier()
    for r in range(K):
        edge_phase()
        plsc.subcore_barrier()
        if r < K - 1:
            row_phase(first=False)
            plsc.subcore_barrier()

    f0 = c * NH + s * RTF
    for off, sz in ((0, NB), (NB, NB), (2 * NB, RTF - 2 * NB)):
        rf = f0 + off
        pltpu.sync_copy(acc_sp.at[pl.ds(rf, sz)], ba.at[pl.ds(0, sz)])
        pltpu.sync_copy(nd.at[pl.ds(rf, sz)], bnd.at[pl.ds(0, sz)])
        pltpu.sync_copy(g0.at[pl.ds(rf, sz)], bg0.at[pl.ds(0, sz)])

        def fbody(k, _):
            bu[k] = bnd[k] * ba[k] + bg0[k]
            return 0

        lax.fori_loop(0, sz, fbody, 0)
        pltpu.sync_copy(bu.at[pl.ds(0, sz)], out.at[pl.ds(rf, sz)])


_fprop_kernel = functools.partial(
    pl.kernel,
    out_type=jax.ShapeDtypeStruct((NPAD, 16), F32),
    mesh=_mesh,
    compiler_params=_sc_params,
    scratch_types=[
        pltpu.VMEM((FNCH, PCH), jnp.int32),
        pltpu.VMEM((FNCH, PCH), jnp.int32),
        pltpu.VMEM((NB, 16), F32),
        pltpu.VMEM((NB, 16), F32),
        pltpu.VMEM((NB, 16), F32),
        pltpu.VMEM((NB, 16), F32),
        pltpu.VMEM((NB, 16), F32),
        pltpu.VMEM((NB, 16), F32),
        pltpu.VMEM((PCH, 16), F32),
        pltpu.VMEM((PCH, 16), F32),
        pltpu.VMEM((PCH, 16), F32),
        pltpu.VMEM_SHARED((NPAD, 16), F32),
        pltpu.VMEM_SHARED((NPAD, 16), F32),
        pltpu.SemaphoreType.DMA,
        pltpu.SemaphoreType.DMA,
        pltpu.SemaphoreType.DMA,
        pltpu.SemaphoreType.DMA,
        pltpu.SemaphoreType.DMA,
        pltpu.SemaphoreType.DMA,
    ],
)(_fprop_body)


def _final_body(f_ref, o_ref):
    o_ref[...] = f_ref[:N]


def _final(feat):
    return pl.pallas_call(
        _final_body,
        out_shape=jax.ShapeDtypeStruct((N, 16), F32),
    )(feat)


def kernel(in_feat, edge_index, W1, b1, W2, b2):
    edge_index = edge_index.astype(jnp.int32)
    srcs_f = edge_index[0].reshape(NS, FNCH, PCH)
    dsts_f = edge_index[1].reshape(NS, FNCH, PCH)
    srcs = edge_index[0].reshape(NW, NCH, CH)
    dsts = edge_index[1].reshape(NW, NCH, CH)

    z0 = _mlp(in_feat, W1, b1, W2, b2)
    degs = _deg_kernel(srcs, dsts)
    nd, ns, g0, a0 = _combine(degs, z0)
    feat = _fprop_kernel(a0, nd, ns, g0, srcs_f, dsts_f)
    return _final(feat)


# speedup vs baseline: 32.4050x; 1.1568x over previous
import functools

import jax
import jax.numpy as jnp
from jax import lax
from jax.experimental import pallas as pl
from jax.experimental.pallas import tpu as pltpu
from jax.experimental.pallas import tpu_sc as plsc

N = 10000
E = 320000
IN_FEATS = 128
H_FEATS = 64
NUM_CLASSES = 16
K = 10
ALPHA = 0.1

NC = 2
NS = 16
NW = NC * NS
NPAD = 10240
RT = NPAD // NS
NB = 128
NNB = RT // NB
EW = E // NW
PCH = 625
PNCH = EW // PCH
CH = 125
NCH = EW // CH
GC = 8
NG = NCH // GC
FNCH = E // NS // PCH
NH = NPAD // 2
RTF = NH // NS
F32 = jnp.float32

_mesh = plsc.VectorSubcoreMesh(
    core_axis_name="c", subcore_axis_name="s", num_cores=NC, num_subcores=NS
)
_sc_params = pltpu.CompilerParams(use_tc_tiling_on_sc=False)


def _mlp_body(x_ref, w1_ref, b1_ref, w2_ref, b2_ref, o_ref):
    x = x_ref[...]
    h = lax.dot_general(x, w1_ref[...], (((1,), (1,)), ((), ())),
                        preferred_element_type=F32)
    h = jnp.maximum(h + b1_ref[...], 0.0)
    o = lax.dot_general(h, w2_ref[...], (((1,), (1,)), ((), ())),
                        preferred_element_type=F32)
    o_ref[...] = o + b2_ref[...]


def _mlp(in_feat, W1, b1, W2, b2):
    blk = 2000
    grid = (N // blk,)
    return pl.pallas_call(
        _mlp_body,
        grid=grid,
        in_specs=[
            pl.BlockSpec((blk, IN_FEATS), lambda i: (i, 0)),
            pl.BlockSpec((H_FEATS, IN_FEATS), lambda i: (0, 0)),
            pl.BlockSpec((1, H_FEATS), lambda i: (0, 0)),
            pl.BlockSpec((NUM_CLASSES, H_FEATS), lambda i: (0, 0)),
            pl.BlockSpec((1, NUM_CLASSES), lambda i: (0, 0)),
        ],
        out_specs=pl.BlockSpec((blk, NUM_CLASSES), lambda i: (i, 0)),
        out_shape=jax.ShapeDtypeStruct((N, NUM_CLASSES), F32),
    )(in_feat, W1, b1.reshape(1, H_FEATS), W2, b2.reshape(1, NUM_CLASSES))


def _deg_body(srcs, dsts, out, sidx, didx, ones, zb, degs_sp, degd_sp):
    c = lax.axis_index("c")
    s = lax.axis_index("s")
    wid = c * NS + s
    row0 = s * RT

    def fill(k, _):
        ones[k] = jnp.ones((16,), F32)
        return 0

    lax.fori_loop(0, CH, fill, 0)

    def zfill(k, _):
        zb[k] = jnp.zeros((16,), F32)
        return 0

    lax.fori_loop(0, NB, zfill, 0)
    for kk in range(NNB):
        pltpu.sync_copy(zb, degs_sp.at[pl.ds(row0 + kk * NB, NB)])
        pltpu.sync_copy(zb, degd_sp.at[pl.ds(row0 + kk * NB, NB)])
    plsc.subcore_barrier()

    def gbody(g, _):
        pltpu.sync_copy(srcs.at[wid, pl.ds(g * GC, GC)], sidx)
        pltpu.sync_copy(dsts.at[wid, pl.ds(g * GC, GC)], didx)

        def ebody(j, _):
            pltpu.sync_copy(ones, degs_sp.at[sidx.at[j]], add=True)
            pltpu.sync_copy(ones, degd_sp.at[didx.at[j]], add=True)
            return 0

        lax.fori_loop(0, GC, ebody, 0)
        return 0

    lax.fori_loop(0, NG, gbody, 0)
    plsc.subcore_barrier()
    pltpu.sync_copy(degs_sp.at[pl.ds(row0, RT)], out.at[c, 0, pl.ds(row0, RT)])
    pltpu.sync_copy(degd_sp.at[pl.ds(row0, RT)], out.at[c, 1, pl.ds(row0, RT)])


_deg_kernel = functools.partial(
    pl.kernel,
    out_type=jax.ShapeDtypeStruct((NC, 2, NPAD, 16), F32),
    mesh=_mesh,
    compiler_params=_sc_params,
    scratch_types=[
        pltpu.VMEM((GC, CH), jnp.int32),
        pltpu.VMEM((GC, CH), jnp.int32),
        pltpu.VMEM((CH, 16), F32),
        pltpu.VMEM((NB, 16), F32),
        pltpu.VMEM_SHARED((NPAD, 16), F32),
        pltpu.VMEM_SHARED((NPAD, 16), F32),
    ],
)(_deg_body)


NRF = NPAD * 16 // 128
NVF = N * 16 // 128


def _comb_body(d_ref, z_ref, nd_ref, nsd_ref, h0_ref, g0_ref, a0_ref):
    ds = d_ref[0, 0] + d_ref[1, 0]
    dd = d_ref[0, 1] + d_ref[1, 1]
    z = jnp.concatenate(
        [z_ref[...], jnp.zeros((NRF - NVF, 128), F32)], axis=0)
    rows = lax.broadcasted_iota(jnp.int32, (NRF, 128), 0)
    valid = rows < NVF
    nsv = jnp.power(jnp.maximum(ds, 1.0), -0.5)
    ndv = jnp.power(jnp.maximum(dd, 1.0), -0.5)
    nd = (1.0 - ALPHA) * ndv
    nd_ref[...] = jnp.where(valid, nd, 0.0)
    nsd_ref[...] = jnp.where(valid, nsv * nd, 0.0)
    g0 = ALPHA * z
    g0_ref[...] = jnp.where(valid, g0, 0.0)
    h0_ref[...] = jnp.where(valid, nsv * g0, 0.0)
    a0_ref[...] = jnp.where(valid, z / ndv, 0.0)


def _combine(degs, z0):
    shp = jax.ShapeDtypeStruct((NRF, 128), F32)
    outs = pl.pallas_call(
        _comb_body,
        out_shape=(shp, shp, shp, shp, shp),
    )(degs.reshape(NC, 2, NRF, 128), z0.reshape(NVF, 128))
    return tuple(o.reshape(NPAD, 16) for o in outs)


def _fprop_body(a0i, nd, nsd, h0, g0, srcs, dsts, out,
                sidx, didx, ba, b1, b2, bu, bz, gb0, gb1, gb2, u_sp,
                acc_sp, gs0, gs1, gs2, ss0, ss1, ss2):
    s = lax.axis_index("s")
    c = lax.axis_index("c")
    row0 = s * RT

    def zfill(k, _):
        bz[k] = jnp.zeros((16,), F32)
        return 0

    lax.fori_loop(0, NB, zfill, 0)
    pltpu.sync_copy(srcs.at[s], sidx)
    pltpu.sync_copy(dsts.at[s], didx)

    def row_phase(first):
        for kk in range(NNB):
            r0 = row0 + kk * NB
            if first:
                pltpu.sync_copy(a0i.at[pl.ds(r0, NB)], ba)
            else:
                pltpu.sync_copy(acc_sp.at[pl.ds(r0, NB)], ba)
            pltpu.sync_copy(nsd.at[pl.ds(r0, NB)], b1)
            pltpu.sync_copy(h0.at[pl.ds(r0, NB)], b2)

            def nbody(k, _):
                bu[k] = b1[k] * ba[k] + b2[k]
                return 0

            lax.fori_loop(0, NB, nbody, 0)
            pltpu.sync_copy(bu, u_sp.at[pl.ds(r0, NB)])
            pltpu.sync_copy(bz, acc_sp.at[pl.ds(r0, NB)])

    def edge_phase():
        bufs = [gb0, gb1, gb2]
        gsems = [gs0, gs1, gs2]
        ssems = [ss0, ss1, ss2]
        gcp = [None, None, None]
        scp = [None, None, None]
        gcp[0] = pltpu.async_copy(u_sp.at[sidx.at[0]], gb0, gs0)
        gcp[1] = pltpu.async_copy(u_sp.at[sidx.at[1]], gb1, gs1)
        for j in range(FNCH):
            b = j % 3
            gcp[b].wait()
            scp[b] = pltpu.async_copy(
                bufs[b], acc_sp.at[didx.at[j]], ssems[b], add=True)
            if j + 2 < FNCH:
                bn = (j + 2) % 3
                if scp[bn] is not None:
                    scp[bn].wait()
                gcp[bn] = pltpu.async_copy(
                    u_sp.at[sidx.at[j + 2]], bufs[bn], gsems[bn])
        scp[(FNCH - 1) % 3].wait()
        scp[(FNCH - 2) % 3].wait()

    row_phase(first=True)
    plsc.subcore_barrier()
    for r in range(K):
        edge_phase()
        plsc.subcore_barrier()
        if r < K - 1:
            row_phase(first=False)
            plsc.subcore_barrier()

    f0 = c * NH + s * RTF
    for off, sz in ((0, NB), (NB, NB), (2 * NB, RTF - 2 * NB)):
        rf = f0 + off
        pltpu.sync_copy(acc_sp.at[pl.ds(rf, sz)], ba.at[pl.ds(0, sz)])
        pltpu.sync_copy(nd.at[pl.ds(rf, sz)], b1.at[pl.ds(0, sz)])
        pltpu.sync_copy(g0.at[pl.ds(rf, sz)], b2.at[pl.ds(0, sz)])

        def fbody(k, _):
            bu[k] = b1[k] * ba[k] + b2[k]
            return 0

        lax.fori_loop(0, sz, fbody, 0)
        pltpu.sync_copy(bu.at[pl.ds(0, sz)], out.at[pl.ds(rf, sz)])


_fprop_kernel = functools.partial(
    pl.kernel,
    out_type=jax.ShapeDtypeStruct((NPAD, 16), F32),
    mesh=_mesh,
    compiler_params=_sc_params,
    scratch_types=[
        pltpu.VMEM((FNCH, PCH), jnp.int32),
        pltpu.VMEM((FNCH, PCH), jnp.int32),
        pltpu.VMEM((NB, 16), F32),
        pltpu.VMEM((NB, 16), F32),
        pltpu.VMEM((NB, 16), F32),
        pltpu.VMEM((NB, 16), F32),
        pltpu.VMEM((NB, 16), F32),
        pltpu.VMEM((PCH, 16), F32),
        pltpu.VMEM((PCH, 16), F32),
        pltpu.VMEM((PCH, 16), F32),
        pltpu.VMEM_SHARED((NPAD, 16), F32),
        pltpu.VMEM_SHARED((NPAD, 16), F32),
        pltpu.SemaphoreType.DMA,
        pltpu.SemaphoreType.DMA,
        pltpu.SemaphoreType.DMA,
        pltpu.SemaphoreType.DMA,
        pltpu.SemaphoreType.DMA,
        pltpu.SemaphoreType.DMA,
    ],
)(_fprop_body)


def _final_body(f_ref, o_ref):
    o_ref[...] = f_ref[:N]


def _final(feat):
    return pl.pallas_call(
        _final_body,
        out_shape=jax.ShapeDtypeStruct((N, 16), F32),
    )(feat)


def kernel(in_feat, edge_index, W1, b1, W2, b2):
    edge_index = edge_index.astype(jnp.int32)
    srcs_f = edge_index[0].reshape(NS, FNCH, PCH)
    dsts_f = edge_index[1].reshape(NS, FNCH, PCH)
    srcs = edge_index[0].reshape(NW, NCH, CH)
    dsts = edge_index[1].reshape(NW, NCH, CH)

    z0 = _mlp(in_feat, W1, b1, W2, b2)
    degs = _deg_kernel(srcs, dsts)
    nd, nsd, h0, g0, a0 = _combine(degs, z0)
    feat = _fprop_kernel(a0, nd, nsd, h0, g0, srcs_f, dsts_f)
    return _final(feat)
